# baseline (speedup 1.0000x reference)
import jax
import jax.numpy as jnp
from jax import lax
from jax.experimental import pallas as pl
from jax.experimental.pallas import tpu as pltpu

N0, N1, K = 4096, 20, 1000
ROWS = N0 * N1
R = 2048


def _onehot_body(values_ref, idx_ref, out_ref):
    off = values_ref[0]
    on = values_ref[1]
    idx = idx_ref[...]
    kk = lax.broadcasted_iota(jnp.int32, (R, K), 1)
    out_ref[...] = jnp.where(kk == idx, on, off)


def kernel(indices, values):
    out2d = pl.pallas_call(
        _onehot_body,
        grid=(ROWS // R,),
        in_specs=[
            pl.BlockSpec(memory_space=pltpu.SMEM),
            pl.BlockSpec((R, 1), lambda i: (i, 0)),
        ],
        out_specs=pl.BlockSpec((R, K), lambda i: (i, 0)),
        out_shape=jax.ShapeDtypeStruct((ROWS, K), jnp.float32),
    )(values, indices.reshape(ROWS, 1))
    return out2d.reshape(N0, N1, K)


# speedup vs baseline: 1.4965x; 1.4965x over previous
import jax
import jax.numpy as jnp
from jax import lax
from jax.experimental import pallas as pl
from jax.experimental.pallas import tpu as pltpu

N0, N1, K = 4096, 20, 1000
B = 64


def _onehot_body(values_ref, idx_ref, out_ref):
    off = values_ref[0]
    on = values_ref[1]
    idx = idx_ref[...]
    kk = lax.broadcasted_iota(jnp.int32, (B, N1, K), 2)
    out_ref[...] = jnp.where(kk == idx, on, off)


def kernel(indices, values):
    return pl.pallas_call(
        _onehot_body,
        grid=(N0 // B,),
        in_specs=[
            pl.BlockSpec(memory_space=pltpu.SMEM),
            pl.BlockSpec((B, N1, 1), lambda i: (i, 0, 0)),
        ],
        out_specs=pl.BlockSpec((B, N1, K), lambda i: (i, 0, 0)),
        out_shape=jax.ShapeDtypeStruct((N0, N1, K), jnp.float32),
    )(values, indices.reshape(N0, N1, 1))


# speedup vs baseline: 1.5112x; 1.0098x over previous
import jax
import jax.numpy as jnp
from jax import lax
from jax.experimental import pallas as pl
from jax.experimental.pallas import tpu as pltpu

N0, N1, K = 4096, 20, 1000
B = 256


def _onehot_body(values_ref, idx_ref, out_ref):
    off = values_ref[0]
    on = values_ref[1]
    idx = idx_ref[...]
    kk = lax.broadcasted_iota(jnp.int32, (B, N1, K), 2)
    out_ref[...] = jnp.where(kk == idx, on, off)


def kernel(indices, values):
    return pl.pallas_call(
        _onehot_body,
        grid=(N0 // B,),
        in_specs=[
            pl.BlockSpec(memory_space=pltpu.SMEM),
            pl.BlockSpec((B, N1, 1), lambda i: (i, 0, 0)),
        ],
        out_specs=pl.BlockSpec((B, N1, K), lambda i: (i, 0, 0)),
        out_shape=jax.ShapeDtypeStruct((N0, N1, K), jnp.float32),
    )(values, indices.reshape(N0, N1, 1))
